# baseline (speedup 1.0000x reference)
import functools
import math

import jax
import jax.numpy as jnp
from jax import lax
from jax.experimental import pallas as pl
from jax.experimental.pallas import tpu as pltpu
from jax.experimental.pallas import tpu_sc as plsc

_D = 128
_S = 200
_B = 1024
_NC = 2
_NS = 16
_NW = _NC * _NS
_ROWS = _B * _S
_RPW = _ROWS // _NW
_CH = _S
_NCH = _RPW // _CH
_NB = 3
_SCALE = math.sqrt(float(_D))

_mesh = plsc.VectorSubcoreMesh(core_axis_name="c", subcore_axis_name="s")


@functools.partial(
    pl.kernel,
    mesh=_mesh,
    out_type=jax.ShapeDtypeStruct((_ROWS, _D), jnp.float32),
    scratch_types=[
        pltpu.VMEM((_RPW,), jnp.int32),
        pltpu.VMEM((_S, _D), jnp.float32),
    ]
    + [pltpu.VMEM((_CH, _D), jnp.float32) for _ in range(_NB)]
    + [pltpu.VMEM((_CH,), jnp.int32)]
    + [pltpu.SemaphoreType.DMA for _ in range(2 * _NB + 1)],
)
def _sc_embed(idx_hbm, p_hbm, table_hbm, out_hbm, idx_v, p_v, *rest):
    bufs = rest[:_NB]
    idx0_v = rest[_NB]
    gsems = rest[_NB + 1:2 * _NB + 1]
    osems = rest[2 * _NB + 1:3 * _NB + 1]
    ssem = rest[3 * _NB + 1]
    wid = lax.axis_index("s") * _NC + lax.axis_index("c")
    base = wid * _RPW
    pltpu.async_copy(idx_hbm.at[pl.ds(base, _RPW)], idx_v, ssem)
    pltpu.async_copy(p_hbm, p_v, ssem)
    pltpu.sync_copy(idx_hbm.at[pl.ds(base, _CH)], idx0_v)
    pltpu.async_copy(
        table_hbm.at[idx0_v.at[pl.ds(0, 104)]],
        bufs[0].at[pl.ds(0, 104)],
        gsems[0],
    )
    pltpu.async_copy(
        table_hbm.at[idx0_v.at[pl.ds(104, _CH - 104)]],
        bufs[0].at[pl.ds(104, _CH - 104)],
        gsems[0],
    )
    pltpu.make_async_copy(idx_hbm.at[pl.ds(base, _RPW)], idx_v, ssem).wait()

    _H1 = 104
    _H2 = _CH - _H1

    def gather_issue(j, b, lo, n):
        pltpu.async_copy(
            table_hbm.at[idx_v.at[pl.ds(j * _CH + lo, n)]],
            bufs[b].at[pl.ds(lo, n)],
            gsems[b],
        )

    def gather_wait(b, lo, n):
        pltpu.make_async_copy(
            table_hbm.at[idx_v.at[pl.ds(0, n)]],
            bufs[b].at[pl.ds(lo, n)],
            gsems[b],
        ).wait()

    def gather_issue_both(j, b):
        gather_issue(j, b, 0, _H1)
        gather_issue(j, b, _H1, _H2)

    def out_issue(j, b, lo, n):
        pltpu.async_copy(
            bufs[b].at[pl.ds(lo, n)],
            out_hbm.at[pl.ds(base + j * _CH + lo, n)],
            osems[b],
        )

    def out_wait(b):
        pltpu.make_async_copy(
            bufs[b].at[pl.ds(0, _H1)], out_hbm.at[pl.ds(0, _H1)], osems[b]
        ).wait()
        pltpu.make_async_copy(
            bufs[b].at[pl.ds(0, _H2)], out_hbm.at[pl.ds(0, _H2)], osems[b]
        ).wait()

    def compute(b, lo, hi):
        @plsc.parallel_loop(lo, hi, unroll=4)
        def row_body(r, _b=b):
            for cc in range(_D // 16):
                sl = pl.ds(cc * 16, 16)
                bufs[_b][r, sl] = bufs[_b][r, sl] * _SCALE + p_v[r, sl]

    for b in range(1, _NB - 1):
        gather_issue_both(b, b)
    pltpu.make_async_copy(p_hbm, p_v, ssem).wait()

    def step(j, b):
        gather_wait(b, 0, _H1)
        compute(b, 0, _H1)
        out_issue(j, b, 0, _H1)
        gather_wait(b, _H1, _H2)
        nb = (b + _NB - 1) % _NB
        g = j + _NB - 1

        @pl.when(g < _NCH)
        def _():
            @pl.when(j >= 1)
            def _():
                out_wait(nb)

            gather_issue_both(g, nb)

        compute(b, _H1, _CH)
        out_issue(j, b, _H1, _H2)

    def outer(o, carry):
        for b in range(_NB):
            step(o * _NB + b, b)
        return carry

    lax.fori_loop(0, _NCH // _NB, outer, 0, unroll=False)
    for j in range((_NCH // _NB) * _NB, _NCH):
        step(j, j % _NB)
    for b in range(_NB):
        out_wait(b)


def kernel(seq, embedding, p):
    idx = seq.reshape(-1).astype(jnp.int32)
    out = _sc_embed(idx, p[:_S], embedding)
    return out.reshape(_B, _S, _D)


# speedup vs baseline: 1.0045x; 1.0045x over previous
import functools
import math

import jax
import jax.numpy as jnp
from jax import lax
from jax.experimental import pallas as pl
from jax.experimental.pallas import tpu as pltpu
from jax.experimental.pallas import tpu_sc as plsc

_D = 128
_S = 200
_B = 1024
_NC = 2
_NS = 16
_NW = _NC * _NS
_ROWS = _B * _S
_RPW = _ROWS // _NW
_CH = _S
_NCH = _RPW // _CH
_NB = 3
_SCALE = math.sqrt(float(_D))

_mesh = plsc.VectorSubcoreMesh(core_axis_name="c", subcore_axis_name="s")


@functools.partial(
    pl.kernel,
    mesh=_mesh,
    out_type=jax.ShapeDtypeStruct((_ROWS, _D), jnp.float32),
    scratch_types=[
        pltpu.VMEM((_RPW,), jnp.int32),
        pltpu.VMEM((_S, _D), jnp.float32),
    ]
    + [pltpu.VMEM((_CH, _D), jnp.float32) for _ in range(_NB)]
    + [pltpu.VMEM((_CH,), jnp.int32)]
    + [pltpu.SemaphoreType.DMA for _ in range(2 * _NB + 1)],
)
def _sc_embed(idx_hbm, p_hbm, table_hbm, out_hbm, idx_v, p_v, *rest):
    bufs = rest[:_NB]
    idx0_v = rest[_NB]
    gsems = rest[_NB + 1:2 * _NB + 1]
    osems = rest[2 * _NB + 1:3 * _NB + 1]
    ssem = rest[3 * _NB + 1]
    wid = lax.axis_index("s") * _NC + lax.axis_index("c")
    base = wid * _RPW
    pltpu.async_copy(idx_hbm.at[pl.ds(base, _RPW)], idx_v, ssem)
    pltpu.async_copy(p_hbm, p_v, ssem)
    pltpu.sync_copy(idx_hbm.at[pl.ds(base, _CH)], idx0_v)
    pltpu.async_copy(
        table_hbm.at[idx0_v.at[pl.ds(0, 104)]],
        bufs[0].at[pl.ds(0, 104)],
        gsems[0],
    )
    pltpu.async_copy(
        table_hbm.at[idx0_v.at[pl.ds(104, _CH - 104)]],
        bufs[0].at[pl.ds(104, _CH - 104)],
        gsems[0],
    )
    pltpu.make_async_copy(idx_hbm.at[pl.ds(base, _RPW)], idx_v, ssem).wait()

    _H1 = 104
    _H2 = _CH - _H1

    def gather_issue(j, b, lo, n):
        pltpu.async_copy(
            table_hbm.at[idx_v.at[pl.ds(j * _CH + lo, n)]],
            bufs[b].at[pl.ds(lo, n)],
            gsems[b],
        )

    def gather_wait(b, lo, n):
        pltpu.make_async_copy(
            table_hbm.at[idx_v.at[pl.ds(0, n)]],
            bufs[b].at[pl.ds(lo, n)],
            gsems[b],
        ).wait()

    def gather_issue_both(j, b):
        gather_issue(j, b, 0, _H1)
        gather_issue(j, b, _H1, _H2)

    def out_issue(j, b, lo, n):
        pltpu.async_copy(
            bufs[b].at[pl.ds(lo, n)],
            out_hbm.at[pl.ds(base + j * _CH + lo, n)],
            osems[b],
        )

    def out_wait(b):
        pltpu.make_async_copy(
            bufs[b].at[pl.ds(0, _H1)], out_hbm.at[pl.ds(0, _H1)], osems[b]
        ).wait()
        pltpu.make_async_copy(
            bufs[b].at[pl.ds(0, _H2)], out_hbm.at[pl.ds(0, _H2)], osems[b]
        ).wait()

    def compute(b, lo, hi):
        @plsc.parallel_loop(lo, hi, unroll=4)
        def row_body(r, _b=b):
            for cc in range(_D // 16):
                sl = pl.ds(cc * 16, 16)
                bufs[_b][r, sl] = bufs[_b][r, sl] * _SCALE + p_v[r, sl]

    for b in range(1, _NB - 1):
        gather_issue_both(b, b)
    pltpu.make_async_copy(p_hbm, p_v, ssem).wait()

    def step(j, b):
        gather_wait(b, 0, _H1)
        compute(b, 0, _H1)
        out_issue(j, b, 0, _H1)
        gather_wait(b, _H1, _H2)
        compute(b, _H1, _CH)
        out_issue(j, b, _H1, _H2)
        nb = (b + _NB - 1) % _NB
        g = j + _NB - 1

        @pl.when(g < _NCH)
        def _():
            @pl.when(j >= 1)
            def _():
                out_wait(nb)

            gather_issue_both(g, nb)

    def outer(o, carry):
        for b in range(_NB):
            step(o * _NB + b, b)
        return carry

    lax.fori_loop(0, _NCH // _NB, outer, 0, unroll=False)
    for j in range((_NCH // _NB) * _NB, _NCH):
        step(j, j % _NB)
    for b in range(_NB):
        out_wait(b)


def kernel(seq, embedding, p):
    idx = seq.reshape(-1).astype(jnp.int32)
    out = _sc_embed(idx, p[:_S], embedding)
    return out.reshape(_B, _S, _D)
